# Initial kernel scaffold; baseline (speedup 1.0000x reference)
import jax
import jax.numpy as jnp
from jax.experimental import pallas as pl


def kernel(pos, x, batch, Wxi0, bxi0, Wxn0, bxn0, g1, b1, Wxi1, bxi1, Wxn1, bxn1, Wf1, gf, bf, Wf2, bf2):
    raise NotImplementedError("write your pallas kernel here")


# speedup vs baseline: 5.7088x; 5.7088x over previous
import functools

import jax
import jax.numpy as jnp
from jax import lax
from jax.experimental import pallas as pl
from jax.experimental.pallas import tpu as pltpu
from jax.experimental.pallas import tpu_sc as plsc

N = 10000
D = 128
K = 20
NCLS = 16

R_KNN = 200
G_KNN = N // R_KNN

NT = 32
CPT = 320
NPAD = NT * CPT
GC = 4
ROWS = GC * K
SHOTS = CPT // GC
NVR = D // 16


def _knn_body(posb_ref, posbt_ref, out_ref):
    pid = pl.program_id(0)
    dx = posb_ref[:, 0:1] - posbt_ref[0:1, :]
    d = dx * dx
    dy = posb_ref[:, 1:2] - posbt_ref[1:2, :]
    d = d + dy * dy
    dz = posb_ref[:, 2:3] - posbt_ref[2:3, :]
    d = d + dz * dz
    col = lax.broadcasted_iota(jnp.int32, (R_KNN, N), 1)
    row = lax.broadcasted_iota(jnp.int32, (R_KNN, N), 0) + pid * R_KNN
    valid = (posb_ref[:, 3:4] == posbt_ref[3:4, :]) & (col != row)
    inf = jnp.float32(jnp.inf)
    d = jnp.where(valid, d, inf)
    big = jnp.int32(N)
    for k in range(K):
        m = jnp.min(d, axis=1, keepdims=True)
        idx = jnp.min(jnp.where(d == m, col, big), axis=1, keepdims=True)
        out_ref[:, k : k + 1] = idx
        d = jnp.where(col == idx, inf, d)


def _knn(pos, batch):
    posb = jnp.concatenate([pos, batch.astype(jnp.float32)[:, None]], axis=1)
    posbt = posb.T
    return pl.pallas_call(
        _knn_body,
        grid=(G_KNN,),
        in_specs=[
            pl.BlockSpec((R_KNN, 4), lambda i: (i, 0)),
            pl.BlockSpec((4, N), lambda i: (0, 0)),
        ],
        out_specs=pl.BlockSpec((R_KNN, K), lambda i: (i, 0)),
        out_shape=jax.ShapeDtypeStruct((N, K), jnp.int32),
    )(posb, posbt)


def _posw(pos_ref, wp_ref):
    return (
        pos_ref[:, 0:1] * wp_ref[0:1, :]
        + pos_ref[:, 1:2] * wp_ref[1:2, :]
        + pos_ref[:, 2:3] * wp_ref[2:3, :]
    )


def _dense0_body(x_ref, pos_ref, wxi_ref, bxi_ref, wp_ref, wf_ref, bxn_ref,
                 u_ref, c_ref):
    x = x_ref[...]
    xw = jnp.dot(x, wxi_ref[...], preferred_element_type=jnp.float32)
    posw = _posw(pos_ref, wp_ref)
    hw = jnp.dot(x, wf_ref[...], preferred_element_type=jnp.float32)
    u_ref[...] = posw + hw
    c_ref[...] = xw + bxi_ref[...] + bxn_ref[...] - posw


def _dense1_body(g_ref, c_ref, pos_ref, g1_ref, b1_ref, wxi_ref, bxi_ref,
                 wp_ref, wf_ref, bxn_ref, u_ref, cout_ref):
    h = g_ref[...] + c_ref[...]
    m = jnp.mean(h, axis=0, keepdims=True)
    v = jnp.mean((h - m) * (h - m), axis=0, keepdims=True)
    a = g1_ref[...] * (h - m) / jnp.sqrt(v + 1e-5) + b1_ref[...]
    a = jnp.maximum(a, 0.0)
    xw = jnp.dot(a, wxi_ref[...], preferred_element_type=jnp.float32)
    posw = _posw(pos_ref, wp_ref)
    hw = jnp.dot(a, wf_ref[...], preferred_element_type=jnp.float32)
    u_ref[...] = posw + hw
    cout_ref[...] = xw + bxi_ref[...] + bxn_ref[...] - posw


def _head_body(g_ref, c_ref, wf1_ref, gf_ref, bf_ref, wf2_ref, bf2_ref,
               out_ref):
    h = g_ref[...] + c_ref[...]
    f = jnp.dot(h, wf1_ref[...], preferred_element_type=jnp.float32)
    m = jnp.mean(f, axis=0, keepdims=True)
    v = jnp.mean((f - m) * (f - m), axis=0, keepdims=True)
    f = gf_ref[...] * (f - m) / jnp.sqrt(v + 1e-5) + bf_ref[...]
    f = jnp.maximum(f, 0.0)
    out_ref[...] = (
        jnp.dot(f, wf2_ref[...], preferred_element_type=jnp.float32)
        + bf2_ref[...]
    )


def _dense0(x, pos, wxi, bxi, wp, wf, bxn):
    return pl.pallas_call(
        _dense0_body,
        out_shape=(
            jax.ShapeDtypeStruct((N, D), jnp.float32),
            jax.ShapeDtypeStruct((N, D), jnp.float32),
        ),
    )(x, pos, wxi, bxi[None, :], wp, wf, bxn[None, :])


def _dense1(g, c, pos, g1, b1, wxi, bxi, wp, wf, bxn):
    return pl.pallas_call(
        _dense1_body,
        out_shape=(
            jax.ShapeDtypeStruct((N, D), jnp.float32),
            jax.ShapeDtypeStruct((N, D), jnp.float32),
        ),
    )(g, c, pos, g1[None, :], b1[None, :], wxi, bxi[None, :], wp, wf,
      bxn[None, :])


def _head(g, c, wf1, gf, bf, wf2, bf2):
    return pl.pallas_call(
        _head_body,
        out_shape=jax.ShapeDtypeStruct((N, NCLS), jnp.float32),
    )(g, c, wf1, gf[None, :], bf[None, :], wf2, bf2[None, :])


@functools.cache
def _make_sc_gather_max():
    mesh = plsc.VectorSubcoreMesh(core_axis_name="c", subcore_axis_name="s")
    return functools.partial(
        pl.kernel,
        out_type=jax.ShapeDtypeStruct((NPAD, D), jnp.float32),
        mesh=mesh,
        scratch_types=[
            pltpu.VMEM((SHOTS, ROWS), jnp.int32),
            pltpu.VMEM((ROWS, D), jnp.float32),
            pltpu.VMEM((ROWS, D), jnp.float32),
            pltpu.VMEM((CPT, D), jnp.float32),
            pltpu.SemaphoreType.DMA,
            pltpu.SemaphoreType.DMA,
        ],
    )(_sc_gather_max_body)


def _sc_gather_max_body(u_hbm, idx_hbm, out_hbm, idx_v, rows0, rows1, out_v,
                        sem0, sem1):
    wid = lax.axis_index("s") * 2 + lax.axis_index("c")
    pltpu.sync_copy(idx_hbm.at[wid], idx_v)
    rows = [rows0, rows1]
    sems = [sem0, sem1]

    def start(shot, b):
        pltpu.make_async_copy(
            u_hbm.at[idx_v.at[shot]], rows[b], sems[b]
        ).start()

    def wait(b):
        pltpu.make_async_copy(
            u_hbm.at[idx_v.at[0]], rows[b], sems[b]
        ).wait()

    def compute(shot, b):
        rv = rows[b]
        for ci in range(GC):
            accs = [rv[ci * K, pl.ds(v * 16, 16)] for v in range(NVR)]
            for k in range(1, K):
                for v in range(NVR):
                    accs[v] = jnp.maximum(
                        accs[v], rv[ci * K + k, pl.ds(v * 16, 16)]
                    )
            c = shot * GC + ci
            for v in range(NVR):
                out_v[c, pl.ds(v * 16, 16)] = accs[v]

    start(0, 0)
    start(1, 1)

    def body(i, carry):
        for b in range(2):
            s = i * 2 + b
            wait(b)
            compute(s, b)

            @pl.when(s + 2 < SHOTS)
            def _():
                start(s + 2, b)
        return carry

    lax.fori_loop(0, SHOTS // 2, body, 0)
    pltpu.sync_copy(out_v, out_hbm.at[pl.ds(wid * CPT, CPT)])


def _gather_max(u, idx_sc):
    return _make_sc_gather_max()(u, idx_sc)


def kernel(pos, x, batch, Wxi0, bxi0, Wxn0, bxn0, g1, b1, Wxi1, bxi1, Wxn1,
           bxn1, Wf1, gf, bf, Wf2, bf2):
    nbr = _knn(pos, batch)
    pad = jnp.zeros((NPAD - N, K), jnp.int32)
    idx_sc = jnp.concatenate([nbr, pad], axis=0).reshape(NT, SHOTS, ROWS)

    u0, c0 = _dense0(x, pos, Wxi0, bxi0, Wxn0[:3], Wxn0[3:], bxn0)
    gm0 = _gather_max(u0, idx_sc)[:N]
    u1, c1 = _dense1(gm0, c0, pos, g1, b1, Wxi1, bxi1, Wxn1[:3], Wxn1[3:],
                     bxn1)
    gm1 = _gather_max(u1, idx_sc)[:N]
    return _head(gm1, c1, Wf1, gf, bf, Wf2, bf2)


# speedup vs baseline: 8.0696x; 1.4135x over previous
import functools

import jax
import jax.numpy as jnp
from jax import lax
from jax.experimental import pallas as pl
from jax.experimental.pallas import tpu as pltpu
from jax.experimental.pallas import tpu_sc as plsc

N = 10000
D = 128
K = 20
NCLS = 16

R_KNN = 200
G_KNN = N // R_KNN
B_KNN = 512
NBLK = (N + B_KNN - 1) // B_KNN
NPC = NBLK * B_KNN
NB = 8
IMAX = 2**31 - 1

NT = 32
CPT = 320
NPAD = NT * CPT
GC = 4
ROWS = GC * K
SHOTS = CPT // GC
NVR = D // 16


def _knn_body(starts_sm, posb_ref, posbt_ref, out_ref, d_ref):
    pid = pl.program_id(0)
    r0 = pid * R_KNN
    r1 = r0 + R_KNN - 1
    bmin = jnp.int32(0)
    bmax = jnp.int32(0)
    for b in range(NB - 1):
        bmin = bmin + (starts_sm[b + 1] <= r0).astype(jnp.int32)
        bmax = bmax + (starts_sm[b + 1] <= r1).astype(jnp.int32)
    lo = starts_sm[bmin]
    hi = starts_sm[bmax + 1]
    minsize = jnp.int32(N)
    for b in range(NB):
        size_b = starts_sm[b + 1] - starts_sm[b]
        pred = (jnp.int32(b) >= bmin) & (jnp.int32(b) <= bmax)
        minsize = jnp.minimum(minsize, jnp.where(pred, size_b, jnp.int32(N)))
    degen = minsize < K + 1
    j_lo = jnp.where(degen, 0, lo // B_KNN)
    j_hi = jnp.where(degen, NBLK, (hi + B_KNN - 1) // B_KNN)

    row = lax.broadcasted_iota(jnp.int32, (R_KNN, B_KNN), 0) + r0
    bcol = posb_ref[:, 3:4]

    def fill(j, mcache):
        c0 = pl.multiple_of(j * B_KNN, B_KNN)
        dx = posb_ref[:, 0:1] - posbt_ref[0:1, pl.ds(c0, B_KNN)]
        d = dx * dx
        dy = posb_ref[:, 1:2] - posbt_ref[1:2, pl.ds(c0, B_KNN)]
        d = d + dy * dy
        dz = posb_ref[:, 2:3] - posbt_ref[2:3, pl.ds(c0, B_KNN)]
        d = d + dz * dz
        col = lax.broadcasted_iota(jnp.int32, (R_KNN, B_KNN), 1) + c0
        valid = (bcol == posbt_ref[3:4, pl.ds(c0, B_KNN)]) & (col != row)
        d = jnp.where(valid, d, jnp.float32(jnp.inf))
        di = lax.bitcast_convert_type(d, jnp.int32)
        d_ref[:, pl.ds(c0, B_KNN)] = di
        blkmin = jnp.min(di, axis=1, keepdims=True)
        lane = lax.broadcasted_iota(jnp.int32, (R_KNN, 128), 1)
        return jnp.where(lane == j, blkmin, mcache)

    mcache = lax.fori_loop(
        j_lo, j_hi, fill, jnp.full((R_KNN, 128), IMAX, jnp.int32)
    )

    for k in range(K):
        m = jnp.min(mcache, axis=1, keepdims=True)

        def extract(j, carry):
            best, mc = carry
            c0 = pl.multiple_of(j * B_KNN, B_KNN)
            blk = d_ref[:, pl.ds(c0, B_KNN)]
            col = lax.broadcasted_iota(jnp.int32, (R_KNN, B_KNN), 1) + c0
            cand = jnp.min(
                jnp.where(blk == m, col, IMAX), axis=1, keepdims=True
            )
            newly = (best == IMAX) & (cand < IMAX)
            sel = jnp.where(newly, cand, IMAX)
            blk2 = jnp.where(col == sel, IMAX, blk)
            d_ref[:, pl.ds(c0, B_KNN)] = blk2
            lane = lax.broadcasted_iota(jnp.int32, (R_KNN, 128), 1)
            mc = jnp.where(
                lane == j, jnp.min(blk2, axis=1, keepdims=True), mc
            )
            return jnp.where(newly, cand, best), mc

        best, mcache = lax.fori_loop(
            j_lo, j_hi, extract, (jnp.full((R_KNN, 1), IMAX, jnp.int32),
                                  mcache)
        )
        out_ref[:, k : k + 1] = best


def _knn(pos, batch):
    batf = batch.astype(jnp.float32)[:, None]
    posb = jnp.concatenate([pos, batf], axis=1)
    posbt = jnp.pad(
        posb.T, ((0, 0), (0, NPC - N)), constant_values=-1.0
    )
    starts = jnp.searchsorted(
        batch, jnp.arange(NB + 1, dtype=batch.dtype)
    ).astype(jnp.int32)
    return pl.pallas_call(
        _knn_body,
        grid_spec=pltpu.PrefetchScalarGridSpec(
            num_scalar_prefetch=1,
            grid=(G_KNN,),
            in_specs=[
                pl.BlockSpec((R_KNN, 4), lambda i, s: (i, 0)),
                pl.BlockSpec((4, NPC), lambda i, s: (0, 0)),
            ],
            out_specs=pl.BlockSpec((R_KNN, K), lambda i, s: (i, 0)),
            scratch_shapes=[pltpu.VMEM((R_KNN, NPC), jnp.int32)],
        ),
        out_shape=jax.ShapeDtypeStruct((N, K), jnp.int32),
    )(starts, posb, posbt)


def _posw(pos_ref, wp_ref):
    return (
        pos_ref[:, 0:1] * wp_ref[0:1, :]
        + pos_ref[:, 1:2] * wp_ref[1:2, :]
        + pos_ref[:, 2:3] * wp_ref[2:3, :]
    )


def _dense0_body(x_ref, pos_ref, wxi_ref, bxi_ref, wp_ref, wf_ref, bxn_ref,
                 u_ref, c_ref):
    x = x_ref[...]
    xw = jnp.dot(x, wxi_ref[...], preferred_element_type=jnp.float32)
    posw = _posw(pos_ref, wp_ref)
    hw = jnp.dot(x, wf_ref[...], preferred_element_type=jnp.float32)
    u_ref[...] = posw + hw
    c_ref[...] = xw + bxi_ref[...] + bxn_ref[...] - posw


def _dense1_body(g_ref, c_ref, pos_ref, g1_ref, b1_ref, wxi_ref, bxi_ref,
                 wp_ref, wf_ref, bxn_ref, u_ref, cout_ref):
    h = g_ref[...] + c_ref[...]
    m = jnp.mean(h, axis=0, keepdims=True)
    v = jnp.mean((h - m) * (h - m), axis=0, keepdims=True)
    a = g1_ref[...] * (h - m) / jnp.sqrt(v + 1e-5) + b1_ref[...]
    a = jnp.maximum(a, 0.0)
    xw = jnp.dot(a, wxi_ref[...], preferred_element_type=jnp.float32)
    posw = _posw(pos_ref, wp_ref)
    hw = jnp.dot(a, wf_ref[...], preferred_element_type=jnp.float32)
    u_ref[...] = posw + hw
    cout_ref[...] = xw + bxi_ref[...] + bxn_ref[...] - posw


def _head_body(g_ref, c_ref, wf1_ref, gf_ref, bf_ref, wf2_ref, bf2_ref,
               out_ref):
    h = g_ref[...] + c_ref[...]
    f = jnp.dot(h, wf1_ref[...], preferred_element_type=jnp.float32)
    m = jnp.mean(f, axis=0, keepdims=True)
    v = jnp.mean((f - m) * (f - m), axis=0, keepdims=True)
    f = gf_ref[...] * (f - m) / jnp.sqrt(v + 1e-5) + bf_ref[...]
    f = jnp.maximum(f, 0.0)
    out_ref[...] = (
        jnp.dot(f, wf2_ref[...], preferred_element_type=jnp.float32)
        + bf2_ref[...]
    )


def _dense0(x, pos, wxi, bxi, wp, wf, bxn):
    return pl.pallas_call(
        _dense0_body,
        out_shape=(
            jax.ShapeDtypeStruct((N, D), jnp.float32),
            jax.ShapeDtypeStruct((N, D), jnp.float32),
        ),
    )(x, pos, wxi, bxi[None, :], wp, wf, bxn[None, :])


def _dense1(g, c, pos, g1, b1, wxi, bxi, wp, wf, bxn):
    return pl.pallas_call(
        _dense1_body,
        out_shape=(
            jax.ShapeDtypeStruct((N, D), jnp.float32),
            jax.ShapeDtypeStruct((N, D), jnp.float32),
        ),
    )(g, c, pos, g1[None, :], b1[None, :], wxi, bxi[None, :], wp, wf,
      bxn[None, :])


def _head(g, c, wf1, gf, bf, wf2, bf2):
    return pl.pallas_call(
        _head_body,
        out_shape=jax.ShapeDtypeStruct((N, NCLS), jnp.float32),
    )(g, c, wf1, gf[None, :], bf[None, :], wf2, bf2[None, :])


@functools.cache
def _make_sc_gather_max():
    mesh = plsc.VectorSubcoreMesh(core_axis_name="c", subcore_axis_name="s")
    return functools.partial(
        pl.kernel,
        out_type=jax.ShapeDtypeStruct((NPAD, D), jnp.float32),
        mesh=mesh,
        scratch_types=[
            pltpu.VMEM((SHOTS, ROWS), jnp.int32),
            pltpu.VMEM((ROWS, D), jnp.float32),
            pltpu.VMEM((ROWS, D), jnp.float32),
            pltpu.VMEM((CPT, D), jnp.float32),
            pltpu.SemaphoreType.DMA,
            pltpu.SemaphoreType.DMA,
        ],
    )(_sc_gather_max_body)


def _sc_gather_max_body(u_hbm, idx_hbm, out_hbm, idx_v, rows0, rows1, out_v,
                        sem0, sem1):
    wid = lax.axis_index("s") * 2 + lax.axis_index("c")
    pltpu.sync_copy(idx_hbm.at[wid], idx_v)
    rows = [rows0, rows1]
    sems = [sem0, sem1]

    def start(shot, b):
        pltpu.make_async_copy(
            u_hbm.at[idx_v.at[shot]], rows[b], sems[b]
        ).start()

    def wait(b):
        pltpu.make_async_copy(
            u_hbm.at[idx_v.at[0]], rows[b], sems[b]
        ).wait()

    def compute(shot, b):
        rv = rows[b]
        for ci in range(GC):
            accs = [rv[ci * K, pl.ds(v * 16, 16)] for v in range(NVR)]
            for k in range(1, K):
                for v in range(NVR):
                    accs[v] = jnp.maximum(
                        accs[v], rv[ci * K + k, pl.ds(v * 16, 16)]
                    )
            c = shot * GC + ci
            for v in range(NVR):
                out_v[c, pl.ds(v * 16, 16)] = accs[v]

    start(0, 0)
    start(1, 1)

    def body(i, carry):
        for b in range(2):
            s = i * 2 + b
            wait(b)
            compute(s, b)

            @pl.when(s + 2 < SHOTS)
            def _():
                start(s + 2, b)
        return carry

    lax.fori_loop(0, SHOTS // 2, body, 0)
    pltpu.sync_copy(out_v, out_hbm.at[pl.ds(wid * CPT, CPT)])


def _gather_max(u, idx_sc):
    return _make_sc_gather_max()(u, idx_sc)


def kernel(pos, x, batch, Wxi0, bxi0, Wxn0, bxn0, g1, b1, Wxi1, bxi1, Wxn1,
           bxn1, Wf1, gf, bf, Wf2, bf2):
    nbr = _knn(pos, batch)
    pad = jnp.zeros((NPAD - N, K), jnp.int32)
    idx_sc = jnp.concatenate([nbr, pad], axis=0).reshape(NT, SHOTS, ROWS)

    u0, c0 = _dense0(x, pos, Wxi0, bxi0, Wxn0[:3], Wxn0[3:], bxn0)
    gm0 = _gather_max(u0, idx_sc)[:N]
    u1, c1 = _dense1(gm0, c0, pos, g1, b1, Wxi1, bxi1, Wxn1[:3], Wxn1[3:],
                     bxn1)
    gm1 = _gather_max(u1, idx_sc)[:N]
    return _head(gm1, c1, Wf1, gf, bf, Wf2, bf2)


# speedup vs baseline: 8.0824x; 1.0016x over previous
import functools

import jax
import jax.numpy as jnp
from jax import lax
from jax.experimental import pallas as pl
from jax.experimental.pallas import tpu as pltpu
from jax.experimental.pallas import tpu_sc as plsc

N = 10000
D = 128
K = 20
NCLS = 16

R_KNN = 200
G_KNN = N // R_KNN
B_KNN = 512
NBLK = (N + B_KNN - 1) // B_KNN
NPC = NBLK * B_KNN
NB = 8
IMAX = 2**31 - 1

NT = 32
CPT = 320
NPAD = NT * CPT
GC = 4
ROWS = GC * K
SHOTS = CPT // GC
NVR = D // 16


def _knn_body(starts_sm, posb_ref, posbt_ref, out_ref, d_ref):
    pid = pl.program_id(0)
    r0 = pid * R_KNN
    r1 = r0 + R_KNN - 1
    bmin = jnp.int32(0)
    bmax = jnp.int32(0)
    for b in range(NB - 1):
        bmin = bmin + (starts_sm[b + 1] <= r0).astype(jnp.int32)
        bmax = bmax + (starts_sm[b + 1] <= r1).astype(jnp.int32)
    lo = starts_sm[bmin]
    hi = starts_sm[bmax + 1]
    minsize = jnp.int32(N)
    for b in range(NB):
        size_b = starts_sm[b + 1] - starts_sm[b]
        pred = (jnp.int32(b) >= bmin) & (jnp.int32(b) <= bmax)
        minsize = jnp.minimum(minsize, jnp.where(pred, size_b, jnp.int32(N)))
    degen = minsize < K + 1
    j_lo = jnp.where(degen, 0, lo // B_KNN)
    j_hi = jnp.where(degen, NBLK, (hi + B_KNN - 1) // B_KNN)

    row = lax.broadcasted_iota(jnp.int32, (R_KNN, B_KNN), 0) + r0
    bcol = posb_ref[:, 3:4]

    def fill(j, mcache):
        c0 = pl.multiple_of(j * B_KNN, B_KNN)
        dx = posb_ref[:, 0:1] - posbt_ref[0:1, pl.ds(c0, B_KNN)]
        d = dx * dx
        dy = posb_ref[:, 1:2] - posbt_ref[1:2, pl.ds(c0, B_KNN)]
        d = d + dy * dy
        dz = posb_ref[:, 2:3] - posbt_ref[2:3, pl.ds(c0, B_KNN)]
        d = d + dz * dz
        col = lax.broadcasted_iota(jnp.int32, (R_KNN, B_KNN), 1) + c0
        valid = (bcol == posbt_ref[3:4, pl.ds(c0, B_KNN)]) & (col != row)
        d = jnp.where(valid, d, jnp.float32(jnp.inf))
        di = lax.bitcast_convert_type(d, jnp.int32)
        d_ref[:, pl.ds(c0, B_KNN)] = di
        blkmin = jnp.min(di, axis=1, keepdims=True)
        lane = lax.broadcasted_iota(jnp.int32, (R_KNN, 128), 1)
        return jnp.where(lane == j, blkmin, mcache)

    mcache = lax.fori_loop(
        j_lo, j_hi, fill, jnp.full((R_KNN, 128), IMAX, jnp.int32)
    )

    for k in range(K):
        m = jnp.min(mcache, axis=1, keepdims=True)

        def extract(j, carry):
            best, mc = carry
            c0 = pl.multiple_of(j * B_KNN, B_KNN)
            blk = d_ref[:, pl.ds(c0, B_KNN)]
            col = lax.broadcasted_iota(jnp.int32, (R_KNN, B_KNN), 1) + c0
            cand = jnp.min(
                jnp.where(blk == m, col, IMAX), axis=1, keepdims=True
            )
            newly = (best == IMAX) & (cand < IMAX)
            sel = jnp.where(newly, cand, IMAX)
            blk2 = jnp.where(col == sel, IMAX, blk)
            d_ref[:, pl.ds(c0, B_KNN)] = blk2
            lane = lax.broadcasted_iota(jnp.int32, (R_KNN, 128), 1)
            mc = jnp.where(
                lane == j, jnp.min(blk2, axis=1, keepdims=True), mc
            )
            return jnp.where(newly, cand, best), mc

        best, mcache = lax.fori_loop(
            j_lo, j_hi, extract, (jnp.full((R_KNN, 1), IMAX, jnp.int32),
                                  mcache)
        )
        out_ref[:, k : k + 1] = best


def _knn(pos, batch):
    batf = batch.astype(jnp.float32)[:, None]
    posb = jnp.concatenate([pos, batf], axis=1)
    posbt = jnp.pad(
        posb.T, ((0, 0), (0, NPC - N)), constant_values=-1.0
    )
    starts = jnp.searchsorted(
        batch, jnp.arange(NB + 1, dtype=batch.dtype)
    ).astype(jnp.int32)
    return pl.pallas_call(
        _knn_body,
        grid_spec=pltpu.PrefetchScalarGridSpec(
            num_scalar_prefetch=1,
            grid=(G_KNN,),
            in_specs=[
                pl.BlockSpec((R_KNN, 4), lambda i, s: (i, 0)),
                pl.BlockSpec((4, NPC), lambda i, s: (0, 0)),
            ],
            out_specs=pl.BlockSpec((R_KNN, K), lambda i, s: (i, 0)),
            scratch_shapes=[pltpu.VMEM((R_KNN, NPC), jnp.int32)],
        ),
        out_shape=jax.ShapeDtypeStruct((N, K), jnp.int32),
    )(starts, posb, posbt)


def _posw(pos_ref, wp_ref):
    return (
        pos_ref[:, 0:1] * wp_ref[0:1, :]
        + pos_ref[:, 1:2] * wp_ref[1:2, :]
        + pos_ref[:, 2:3] * wp_ref[2:3, :]
    )


def _dense0_body(x_ref, pos_ref, wxi_ref, bxi_ref, wp_ref, wf_ref, bxn_ref,
                 u_ref, c_ref):
    x = x_ref[...]
    xw = jnp.dot(x, wxi_ref[...], preferred_element_type=jnp.float32)
    posw = _posw(pos_ref, wp_ref)
    hw = jnp.dot(x, wf_ref[...], preferred_element_type=jnp.float32)
    u_ref[...] = posw + hw
    c_ref[...] = xw + bxi_ref[...] + bxn_ref[...] - posw


def _dense1_body(g_ref, c_ref, pos_ref, g1_ref, b1_ref, wxi_ref, bxi_ref,
                 wp_ref, wf_ref, bxn_ref, u_ref, cout_ref):
    h = g_ref[...] + c_ref[...]
    m = jnp.mean(h, axis=0, keepdims=True)
    v = jnp.mean((h - m) * (h - m), axis=0, keepdims=True)
    a = g1_ref[...] * (h - m) / jnp.sqrt(v + 1e-5) + b1_ref[...]
    a = jnp.maximum(a, 0.0)
    xw = jnp.dot(a, wxi_ref[...], preferred_element_type=jnp.float32)
    posw = _posw(pos_ref, wp_ref)
    hw = jnp.dot(a, wf_ref[...], preferred_element_type=jnp.float32)
    u_ref[...] = posw + hw
    cout_ref[...] = xw + bxi_ref[...] + bxn_ref[...] - posw


def _head_body(g_ref, c_ref, wf1_ref, gf_ref, bf_ref, wf2_ref, bf2_ref,
               out_ref):
    h = g_ref[...] + c_ref[...]
    f = jnp.dot(h, wf1_ref[...], preferred_element_type=jnp.float32)
    m = jnp.mean(f, axis=0, keepdims=True)
    v = jnp.mean((f - m) * (f - m), axis=0, keepdims=True)
    f = gf_ref[...] * (f - m) / jnp.sqrt(v + 1e-5) + bf_ref[...]
    f = jnp.maximum(f, 0.0)
    out_ref[...] = (
        jnp.dot(f, wf2_ref[...], preferred_element_type=jnp.float32)
        + bf2_ref[...]
    )


def _dense0(x, pos, wxi, bxi, wp, wf, bxn):
    return pl.pallas_call(
        _dense0_body,
        out_shape=(
            jax.ShapeDtypeStruct((N, D), jnp.float32),
            jax.ShapeDtypeStruct((N, D), jnp.float32),
        ),
    )(x, pos, wxi, bxi[None, :], wp, wf, bxn[None, :])


def _dense1(g, c, pos, g1, b1, wxi, bxi, wp, wf, bxn):
    return pl.pallas_call(
        _dense1_body,
        out_shape=(
            jax.ShapeDtypeStruct((N, D), jnp.float32),
            jax.ShapeDtypeStruct((N, D), jnp.float32),
        ),
    )(g, c, pos, g1[None, :], b1[None, :], wxi, bxi[None, :], wp, wf,
      bxn[None, :])


def _head(g, c, wf1, gf, bf, wf2, bf2):
    return pl.pallas_call(
        _head_body,
        out_shape=jax.ShapeDtypeStruct((N, NCLS), jnp.float32),
    )(g, c, wf1, gf[None, :], bf[None, :], wf2, bf2[None, :])


@functools.cache
def _make_sc_gather_max():
    mesh = plsc.VectorSubcoreMesh(core_axis_name="c", subcore_axis_name="s")
    return functools.partial(
        pl.kernel,
        out_type=jax.ShapeDtypeStruct((NPAD, D), jnp.float32),
        mesh=mesh,
        scratch_types=[
            pltpu.VMEM((SHOTS, ROWS), jnp.int32),
            pltpu.VMEM((ROWS, D), jnp.float32),
            pltpu.VMEM((ROWS, D), jnp.float32),
            pltpu.VMEM((ROWS, D), jnp.float32),
            pltpu.VMEM((ROWS, D), jnp.float32),
            pltpu.VMEM((CPT, D), jnp.float32),
            pltpu.SemaphoreType.DMA,
            pltpu.SemaphoreType.DMA,
            pltpu.SemaphoreType.DMA,
            pltpu.SemaphoreType.DMA,
        ],
    )(_sc_gather_max_body)


NBUF = 4


def _sc_gather_max_body(u_hbm, idx_hbm, out_hbm, idx_v, rows0, rows1, rows2,
                        rows3, out_v, sem0, sem1, sem2, sem3):
    wid = lax.axis_index("s") * 2 + lax.axis_index("c")
    pltpu.sync_copy(idx_hbm.at[wid], idx_v)
    rows = [rows0, rows1, rows2, rows3]
    sems = [sem0, sem1, sem2, sem3]

    def start(shot, b):
        pltpu.make_async_copy(
            u_hbm.at[idx_v.at[shot]], rows[b], sems[b]
        ).start()

    def wait(b):
        pltpu.make_async_copy(
            u_hbm.at[idx_v.at[0]], rows[b], sems[b]
        ).wait()

    def compute(shot, b):
        rv = rows[b]
        for ci in range(GC):
            accs = [rv[ci * K, pl.ds(v * 16, 16)] for v in range(NVR)]
            for k in range(1, K):
                for v in range(NVR):
                    accs[v] = jnp.maximum(
                        accs[v], rv[ci * K + k, pl.ds(v * 16, 16)]
                    )
            c = shot * GC + ci
            for v in range(NVR):
                out_v[c, pl.ds(v * 16, 16)] = accs[v]

    for b in range(NBUF):
        start(b, b)

    def body(i, carry):
        for b in range(NBUF):
            s = i * NBUF + b
            wait(b)
            compute(s, b)

            @pl.when(s + NBUF < SHOTS)
            def _():
                start(s + NBUF, b)
        return carry

    lax.fori_loop(0, SHOTS // NBUF, body, 0)
    pltpu.sync_copy(out_v, out_hbm.at[pl.ds(wid * CPT, CPT)])


def _gather_max(u, idx_sc):
    return _make_sc_gather_max()(u, idx_sc)


def kernel(pos, x, batch, Wxi0, bxi0, Wxn0, bxn0, g1, b1, Wxi1, bxi1, Wxn1,
           bxn1, Wf1, gf, bf, Wf2, bf2):
    nbr = _knn(pos, batch)
    pad = jnp.zeros((NPAD - N, K), jnp.int32)
    idx_sc = jnp.concatenate([nbr, pad], axis=0).reshape(NT, SHOTS, ROWS)

    u0, c0 = _dense0(x, pos, Wxi0, bxi0, Wxn0[:3], Wxn0[3:], bxn0)
    gm0 = _gather_max(u0, idx_sc)[:N]
    u1, c1 = _dense1(gm0, c0, pos, g1, b1, Wxi1, bxi1, Wxn1[:3], Wxn1[3:],
                     bxn1)
    gm1 = _gather_max(u1, idx_sc)[:N]
    return _head(gm1, c1, Wf1, gf, bf, Wf2, bf2)
